# baseline (speedup 1.0000x reference)
import functools

import jax
import jax.numpy as jnp
from jax import lax
from jax.experimental import pallas as pl
from jax.experimental.pallas import tpu as pltpu
from jax.experimental.pallas import tpu_sc as plsc

N = 10000
E = 320000
H = 64
EPS = 1e-5

ROWS = 2000
GRID = N // ROWS

NW = 32
CHUNK = 128
E_PAD = 327680
EDGES_PER_W = E_PAD // NW
NCHUNK = EDGES_PER_W // CHUNK
AGG_ROWS = N + 8
SLICE = 624
SLICE_LAST = N - 15 * SLICE
ZROWS = 160


def _segsum_body(hw, edges, out, sidx, didx, rows0, rows1, rows2, rows3,
                 zbuf, agg, gsem0, gsem1, gsem2, gsem3,
                 ssem0, ssem1, ssem2, ssem3):
    cid = lax.axis_index("c")
    sid = lax.axis_index("s")
    wid = cid * 16 + sid

    pltpu.sync_copy(edges.at[0, wid], sidx)
    pltpu.sync_copy(edges.at[1, wid], didx)

    rows = (rows0, rows1, rows2, rows3)
    gsem = (gsem0, gsem1, gsem2, gsem3)
    ssem = (ssem0, ssem1, ssem2, ssem3)

    for b in range(3):
        pltpu.async_copy(hw.at[sidx.at[b]], rows[b], gsem[b])

    def zrow(i, _):
        for k in range(4):
            zbuf[i, pl.ds(k * 16, 16)] = jnp.zeros((16,), jnp.float32)
        return 0
    lax.fori_loop(0, ZROWS, zrow, 0)

    base = sid * SLICE
    for k in range(3):
        pltpu.sync_copy(zbuf, agg.at[pl.ds(base + k * ZROWS, ZROWS)])

    @pl.when(sid < 15)
    def _():
        pltpu.sync_copy(zbuf.at[pl.ds(0, SLICE - 3 * ZROWS)],
                        agg.at[pl.ds(base + 3 * ZROWS, SLICE - 3 * ZROWS)])

    @pl.when(sid == 15)
    def _():
        pltpu.sync_copy(zbuf, agg.at[pl.ds(base + 3 * ZROWS, ZROWS)])

    plsc.subcore_barrier()

    def step(s, _):
        for b in range(4):
            jj = 4 * s + b
            pltpu.make_async_copy(hw.at[sidx.at[jj]], rows[b], gsem[b]).wait()
            pltpu.async_copy(rows[b], agg.at[didx.at[jj]], ssem[b], add=True)
            b3 = (b + 3) % 4

            @pl.when(jj + 3 < NCHUNK)
            def _():
                @pl.when(jj >= 1)
                def _():
                    pltpu.make_async_copy(
                        rows[b3], agg.at[didx.at[jj - 1]], ssem[b3]).wait()
                pltpu.async_copy(hw.at[sidx.at[jj + 3]], rows[b3], gsem[b3])
        return 0
    lax.fori_loop(0, NCHUNK // 4, step, 0)

    for b in range(4):
        pltpu.make_async_copy(rows[b], agg.at[didx.at[NCHUNK - 4 + b]],
                              ssem[b]).wait()

    plsc.subcore_barrier()

    @pl.when(sid < 15)
    def _():
        pltpu.sync_copy(agg.at[pl.ds(sid * SLICE, SLICE)],
                        out.at[pl.ds(sid * SLICE, SLICE),
                               pl.ds(cid * H, H)])

    @pl.when(sid == 15)
    def _():
        pltpu.sync_copy(agg.at[pl.ds(15 * SLICE, SLICE_LAST)],
                        out.at[pl.ds(15 * SLICE, SLICE_LAST),
                               pl.ds(cid * H, H)])


@functools.cache
def _make_segsum():
    return pl.kernel(
        _segsum_body,
        out_type=jax.ShapeDtypeStruct((N, 2 * H), jnp.float32),
        mesh=plsc.VectorSubcoreMesh(core_axis_name="c", subcore_axis_name="s",
                                    num_cores=2, num_subcores=16),
        scratch_types=[
            pltpu.VMEM((NCHUNK, CHUNK), jnp.int32),
            pltpu.VMEM((NCHUNK, CHUNK), jnp.int32),
            pltpu.VMEM((CHUNK, H), jnp.float32),
            pltpu.VMEM((CHUNK, H), jnp.float32),
            pltpu.VMEM((CHUNK, H), jnp.float32),
            pltpu.VMEM((CHUNK, H), jnp.float32),
            pltpu.VMEM((ZROWS, H), jnp.float32),
            pltpu.VMEM_SHARED((AGG_ROWS, H), jnp.float32),
        ] + [pltpu.SemaphoreType.DMA] * 8,
        compiler_params=pltpu.CompilerParams(use_tc_tiling_on_sc=False),
    )


def _segsum(hw, edges):
    return _make_segsum()(hw, edges)


def _dense0_body(h_ref, W_ref, b_ref, Wr_ref, br_ref, hw_ref, res_ref):
    h = h_ref[...]
    hw_ref[...] = jnp.dot(h, W_ref[...],
                          preferred_element_type=jnp.float32) + b_ref[...]
    r = jnp.dot(h, Wr_ref[...], preferred_element_type=jnp.float32) + br_ref[...]
    res_ref[...] = jnp.maximum(r, 0.0)


def _accum_u(parts_ref, res_ref, u_all, st, i):
    p = parts_ref[...]
    u = jnp.maximum(p[:, :H] + p[:, H:], 0.0) + res_ref[...]
    u_all[pl.ds(i * ROWS, ROWS), :] = u

    @pl.when(i == 0)
    def _():
        st[...] = jnp.zeros_like(st)

    s = jnp.sum(u, axis=0, keepdims=True)
    s2 = jnp.sum(u * u, axis=0, keepdims=True)
    st[...] += jnp.concatenate([s, s2], axis=0)


def _norm_coeffs(st, g_ref, be_ref):
    mean = st[0:1, :] * (1.0 / N)
    var = st[1:2, :] * (1.0 / N) - mean * mean
    a = g_ref[...] * lax.rsqrt(var + EPS)
    c = be_ref[...] - mean * a
    return a, c


def _combdense_body(parts_ref, res_ref, g_ref, be_ref, W_ref, b_ref, Wr_ref,
                    br_ref, hw_ref, res_out, u_all, st):
    p = pl.program_id(0)
    i = pl.program_id(1)

    @pl.when(p == 0)
    def _():
        _accum_u(parts_ref, res_ref, u_all, st, i)

    @pl.when(p == 1)
    def _():
        a, c = _norm_coeffs(st, g_ref, be_ref)
        h = u_all[pl.ds(i * ROWS, ROWS), :] * a + c
        hw_ref[...] = jnp.dot(h, W_ref[...],
                              preferred_element_type=jnp.float32) + b_ref[...]
        r = jnp.dot(h, Wr_ref[...],
                    preferred_element_type=jnp.float32) + br_ref[...]
        res_out[...] = jnp.maximum(r, 0.0)


def _combhead_body(parts_ref, res_ref, g_ref, be_ref, Wd_ref, bd_ref,
                   out_ref, u_all, st):
    p = pl.program_id(0)
    i = pl.program_id(1)

    @pl.when(p == 0)
    def _():
        _accum_u(parts_ref, res_ref, u_all, st, i)

    @pl.when(p == 1)
    def _():
        a, c = _norm_coeffs(st, g_ref, be_ref)
        h = u_all[pl.ds(i * ROWS, ROWS), :] * a + c
        logits = jnp.dot(h, Wd_ref[...],
                         preferred_element_type=jnp.float32) + bd_ref[...]
        z = logits - jnp.max(logits, axis=1, keepdims=True)
        ez = jnp.exp(z)
        out_ref[...] = ez / jnp.sum(ez, axis=1, keepdims=True)


def _row_spec(d):
    return pl.BlockSpec((ROWS, d), lambda i: (i, 0))


def _full_spec(r, d):
    return pl.BlockSpec((r, d), lambda i: (0, 0))


def _full_spec2(r, d):
    return pl.BlockSpec((r, d), lambda p, i: (0, 0))


def _phase0_parts_spec():
    return pl.BlockSpec((ROWS, 2 * H),
                        lambda p, i: (jnp.where(p == 0, i, GRID - 1), 0))


def _phase0_row_spec(d):
    return pl.BlockSpec((ROWS, d),
                        lambda p, i: (jnp.where(p == 0, i, GRID - 1), 0))


def _phase1_out_spec(d):
    return pl.BlockSpec((ROWS, d),
                        lambda p, i: (jnp.where(p == 0, 0, i), 0))


def _dense0(h, W, b, Wr, br):
    return pl.pallas_call(
        _dense0_body,
        grid=(GRID,),
        in_specs=[_row_spec(h.shape[1]), _full_spec(*W.shape), _full_spec(1, H),
                  _full_spec(*Wr.shape), _full_spec(1, H)],
        out_specs=[_row_spec(H), _row_spec(H)],
        out_shape=[jax.ShapeDtypeStruct((N, H), jnp.float32)] * 2,
    )(h, W, b, Wr, br)


def _combdense(parts, res, g, be, W, b, Wr, br):
    return pl.pallas_call(
        _combdense_body,
        grid=(2, GRID),
        in_specs=[_phase0_parts_spec(), _phase0_row_spec(H), _full_spec2(1, H),
                  _full_spec2(1, H), _full_spec2(*W.shape), _full_spec2(1, H),
                  _full_spec2(*Wr.shape), _full_spec2(1, H)],
        out_specs=[_phase1_out_spec(H), _phase1_out_spec(H)],
        out_shape=[jax.ShapeDtypeStruct((N, H), jnp.float32)] * 2,
        scratch_shapes=[pltpu.VMEM((N, H), jnp.float32),
                        pltpu.VMEM((2, H), jnp.float32)],
    )(parts, res, g, be, W, b, Wr, br)


def _combhead(parts, res, g, be, Wd, bd):
    return pl.pallas_call(
        _combhead_body,
        grid=(2, GRID),
        in_specs=[_phase0_parts_spec(), _phase0_row_spec(H), _full_spec2(1, H),
                  _full_spec2(1, H), _full_spec2(*Wd.shape), _full_spec2(1, 2)],
        out_specs=_phase1_out_spec(2),
        out_shape=jax.ShapeDtypeStruct((N, 2), jnp.float32),
        scratch_shapes=[pltpu.VMEM((N, H), jnp.float32),
                        pltpu.VMEM((2, H), jnp.float32)],
    )(parts, res, g, be, Wd, bd)


def kernel(in_feat, edge_index, W0, b0, Wr0, br0, g0, be0,
           W1, b1, Wr1, br1, g1, be1, Wd, bd):
    pad = jnp.broadcast_to(jnp.array([[0], [N]], jnp.int32), (2, E_PAD - E))
    edges = jnp.concatenate([edge_index, pad], axis=1)
    edges = edges.reshape(2, NW, NCHUNK, CHUNK)
    b0r, br0r, g0r, be0r = (v.reshape(1, H) for v in (b0, br0, g0, be0))
    b1r, br1r, g1r, be1r = (v.reshape(1, H) for v in (b1, br1, g1, be1))
    bdr = bd.reshape(1, 2)

    hw0, res0 = _dense0(in_feat, W0, b0r, Wr0, br0r)
    parts0 = _segsum(hw0, edges)
    hw1, res1 = _combdense(parts0, res0, g0r, be0r, W1, b1r, Wr1, br1r)
    parts1 = _segsum(hw1, edges)
    return _combhead(parts1, res1, g1r, be1r, Wd, bdr)


# speedup vs baseline: 2.9025x; 2.9025x over previous
import functools

import jax
import jax.numpy as jnp
from jax import lax
from jax.experimental import pallas as pl
from jax.experimental.pallas import tpu as pltpu
from jax.experimental.pallas import tpu_sc as plsc

N = 10000
E = 320000
H = 64
EPS = 1e-5

ROWS = 2000
GRID = N // ROWS

NW = 32
CHUNK = 80
EDGES_PER_W = E // NW
NCHUNK = EDGES_PER_W // CHUNK
AGG_ROWS = N
SLICE = 624
SLICE_LAST = N - 15 * SLICE
ZROWS = 160


def _segsum_body(hw, edges, out, sidx, didx, rows0, rows1, rows2, rows3,
                 zbuf, agg, gsem0, gsem1, gsem2, gsem3,
                 ssem0, ssem1, ssem2, ssem3):
    cid = lax.axis_index("c")
    sid = lax.axis_index("s")
    wid = cid * 16 + sid

    pltpu.sync_copy(edges.at[0, wid], sidx)
    pltpu.sync_copy(edges.at[1, wid], didx)

    rows = (rows0, rows1, rows2, rows3)
    gsem = (gsem0, gsem1, gsem2, gsem3)
    ssem = (ssem0, ssem1, ssem2, ssem3)

    for b in range(3):
        pltpu.async_copy(hw.at[sidx.at[b]], rows[b], gsem[b])

    def zrow(i, _):
        for k in range(4):
            zbuf[i, pl.ds(k * 16, 16)] = jnp.zeros((16,), jnp.float32)
        return 0
    lax.fori_loop(0, ZROWS, zrow, 0)

    base = sid * SLICE
    for k in range(3):
        pltpu.sync_copy(zbuf, agg.at[pl.ds(base + k * ZROWS, ZROWS)])

    @pl.when(sid < 15)
    def _():
        pltpu.sync_copy(zbuf.at[pl.ds(0, SLICE - 3 * ZROWS)],
                        agg.at[pl.ds(base + 3 * ZROWS, SLICE - 3 * ZROWS)])

    @pl.when(sid == 15)
    def _():
        pltpu.sync_copy(zbuf, agg.at[pl.ds(base + 3 * ZROWS, ZROWS)])

    plsc.subcore_barrier()

    def step(s, _):
        for b in range(4):
            jj = 4 * s + b
            pltpu.make_async_copy(hw.at[sidx.at[jj]], rows[b], gsem[b]).wait()
            pltpu.async_copy(rows[b], agg.at[didx.at[jj]], ssem[b], add=True)
            b3 = (b + 3) % 4

            @pl.when(jj + 3 < NCHUNK)
            def _():
                @pl.when(jj >= 1)
                def _():
                    pltpu.make_async_copy(
                        rows[b3], agg.at[didx.at[jj - 1]], ssem[b3]).wait()
                pltpu.async_copy(hw.at[sidx.at[jj + 3]], rows[b3], gsem[b3])
        return 0
    lax.fori_loop(0, (NCHUNK - 1) // 4, step, 0)

    last = NCHUNK - 1
    pltpu.make_async_copy(hw.at[sidx.at[last]], rows[0], gsem[0]).wait()
    pltpu.async_copy(rows[0], agg.at[didx.at[last]], ssem[0], add=True)
    pltpu.make_async_copy(rows[0], agg.at[didx.at[last]], ssem[0]).wait()
    pltpu.make_async_copy(rows[1], agg.at[didx.at[NCHUNK - 4]], ssem[1]).wait()
    pltpu.make_async_copy(rows[2], agg.at[didx.at[NCHUNK - 3]], ssem[2]).wait()
    pltpu.make_async_copy(rows[3], agg.at[didx.at[NCHUNK - 2]], ssem[3]).wait()

    plsc.subcore_barrier()

    @pl.when(sid < 15)
    def _():
        pltpu.sync_copy(agg.at[pl.ds(sid * SLICE, SLICE)],
                        out.at[pl.ds(sid * SLICE, SLICE),
                               pl.ds(cid * H, H)])

    @pl.when(sid == 15)
    def _():
        pltpu.sync_copy(agg.at[pl.ds(15 * SLICE, SLICE_LAST)],
                        out.at[pl.ds(15 * SLICE, SLICE_LAST),
                               pl.ds(cid * H, H)])


@functools.cache
def _make_segsum():
    return pl.kernel(
        _segsum_body,
        out_type=jax.ShapeDtypeStruct((N, 2 * H), jnp.float32),
        mesh=plsc.VectorSubcoreMesh(core_axis_name="c", subcore_axis_name="s",
                                    num_cores=2, num_subcores=16),
        scratch_types=[
            pltpu.VMEM((NCHUNK, CHUNK), jnp.int32),
            pltpu.VMEM((NCHUNK, CHUNK), jnp.int32),
            pltpu.VMEM((CHUNK, H), jnp.float32),
            pltpu.VMEM((CHUNK, H), jnp.float32),
            pltpu.VMEM((CHUNK, H), jnp.float32),
            pltpu.VMEM((CHUNK, H), jnp.float32),
            pltpu.VMEM((ZROWS, H), jnp.float32),
            pltpu.VMEM_SHARED((AGG_ROWS, H), jnp.float32),
        ] + [pltpu.SemaphoreType.DMA] * 8,
        compiler_params=pltpu.CompilerParams(use_tc_tiling_on_sc=False),
    )


def _segsum(hw, edges):
    return _make_segsum()(hw, edges)


def _dense0_body(h_ref, W_ref, b_ref, Wr_ref, br_ref, hw_ref, res_ref):
    h = h_ref[...]
    hw_ref[...] = jnp.dot(h, W_ref[...],
                          preferred_element_type=jnp.float32) + b_ref[...]
    r = jnp.dot(h, Wr_ref[...], preferred_element_type=jnp.float32) + br_ref[...]
    res_ref[...] = jnp.maximum(r, 0.0)


def _accum_u(parts_ref, res_ref, u_all, st, i):
    p = parts_ref[...]
    u = jnp.maximum(p[:, :H] + p[:, H:], 0.0) + res_ref[...]
    u_all[pl.ds(i * ROWS, ROWS), :] = u

    @pl.when(i == 0)
    def _():
        st[...] = jnp.zeros_like(st)

    s = jnp.sum(u, axis=0, keepdims=True)
    s2 = jnp.sum(u * u, axis=0, keepdims=True)
    st[...] += jnp.concatenate([s, s2], axis=0)


def _norm_coeffs(st, g_ref, be_ref):
    mean = st[0:1, :] * (1.0 / N)
    var = st[1:2, :] * (1.0 / N) - mean * mean
    a = g_ref[...] * lax.rsqrt(var + EPS)
    c = be_ref[...] - mean * a
    return a, c


def _combdense_body(parts_ref, res_ref, g_ref, be_ref, W_ref, b_ref, Wr_ref,
                    br_ref, hw_ref, res_out, u_all, st):
    p = pl.program_id(0)
    i = pl.program_id(1)

    @pl.when(p == 0)
    def _():
        _accum_u(parts_ref, res_ref, u_all, st, i)

    @pl.when(p == 1)
    def _():
        a, c = _norm_coeffs(st, g_ref, be_ref)
        h = u_all[pl.ds(i * ROWS, ROWS), :] * a + c
        hw_ref[...] = jnp.dot(h, W_ref[...],
                              preferred_element_type=jnp.float32) + b_ref[...]
        r = jnp.dot(h, Wr_ref[...],
                    preferred_element_type=jnp.float32) + br_ref[...]
        res_out[...] = jnp.maximum(r, 0.0)


def _combhead_body(parts_ref, res_ref, g_ref, be_ref, Wd_ref, bd_ref,
                   out_ref, u_all, st):
    p = pl.program_id(0)
    i = pl.program_id(1)

    @pl.when(p == 0)
    def _():
        _accum_u(parts_ref, res_ref, u_all, st, i)

    @pl.when(p == 1)
    def _():
        a, c = _norm_coeffs(st, g_ref, be_ref)
        h = u_all[pl.ds(i * ROWS, ROWS), :] * a + c
        logits = jnp.dot(h, Wd_ref[...],
                         preferred_element_type=jnp.float32) + bd_ref[...]
        z = logits - jnp.max(logits, axis=1, keepdims=True)
        ez = jnp.exp(z)
        out_ref[...] = ez / jnp.sum(ez, axis=1, keepdims=True)


def _row_spec(d):
    return pl.BlockSpec((ROWS, d), lambda i: (i, 0))


def _full_spec(r, d):
    return pl.BlockSpec((r, d), lambda i: (0, 0))


def _full_spec2(r, d):
    return pl.BlockSpec((r, d), lambda p, i: (0, 0))


def _phase0_parts_spec():
    return pl.BlockSpec((ROWS, 2 * H),
                        lambda p, i: (jnp.where(p == 0, i, GRID - 1), 0))


def _phase0_row_spec(d):
    return pl.BlockSpec((ROWS, d),
                        lambda p, i: (jnp.where(p == 0, i, GRID - 1), 0))


def _phase1_out_spec(d):
    return pl.BlockSpec((ROWS, d),
                        lambda p, i: (jnp.where(p == 0, 0, i), 0))


def _dense0(h, W, b, Wr, br):
    return pl.pallas_call(
        _dense0_body,
        grid=(GRID,),
        in_specs=[_row_spec(h.shape[1]), _full_spec(*W.shape), _full_spec(1, H),
                  _full_spec(*Wr.shape), _full_spec(1, H)],
        out_specs=[_row_spec(H), _row_spec(H)],
        out_shape=[jax.ShapeDtypeStruct((N, H), jnp.float32)] * 2,
    )(h, W, b, Wr, br)


def _combdense(parts, res, g, be, W, b, Wr, br):
    return pl.pallas_call(
        _combdense_body,
        grid=(2, GRID),
        in_specs=[_phase0_parts_spec(), _phase0_row_spec(H), _full_spec2(1, H),
                  _full_spec2(1, H), _full_spec2(*W.shape), _full_spec2(1, H),
                  _full_spec2(*Wr.shape), _full_spec2(1, H)],
        out_specs=[_phase1_out_spec(H), _phase1_out_spec(H)],
        out_shape=[jax.ShapeDtypeStruct((N, H), jnp.float32)] * 2,
        scratch_shapes=[pltpu.VMEM((N, H), jnp.float32),
                        pltpu.VMEM((2, H), jnp.float32)],
    )(parts, res, g, be, W, b, Wr, br)


def _combhead(parts, res, g, be, Wd, bd):
    return pl.pallas_call(
        _combhead_body,
        grid=(2, GRID),
        in_specs=[_phase0_parts_spec(), _phase0_row_spec(H), _full_spec2(1, H),
                  _full_spec2(1, H), _full_spec2(*Wd.shape), _full_spec2(1, 2)],
        out_specs=_phase1_out_spec(2),
        out_shape=jax.ShapeDtypeStruct((N, 2), jnp.float32),
        scratch_shapes=[pltpu.VMEM((N, H), jnp.float32),
                        pltpu.VMEM((2, H), jnp.float32)],
    )(parts, res, g, be, Wd, bd)


def kernel(in_feat, edge_index, W0, b0, Wr0, br0, g0, be0,
           W1, b1, Wr1, br1, g1, be1, Wd, bd):
    edges = edge_index.reshape(2, NW, NCHUNK, CHUNK)
    b0r, br0r, g0r, be0r = (v.reshape(1, H) for v in (b0, br0, g0, be0))
    b1r, br1r, g1r, be1r = (v.reshape(1, H) for v in (b1, br1, g1, be1))
    bdr = bd.reshape(1, 2)

    hw0, res0 = _dense0(in_feat, W0, b0r, Wr0, br0r)
    parts0 = _segsum(hw0, edges)
    hw1, res1 = _combdense(parts0, res0, g0r, be0r, W1, b1r, Wr1, br1r)
    parts1 = _segsum(hw1, edges)
    return _combhead(parts1, res1, g1r, be1r, Wd, bdr)
